# Initial kernel scaffold; baseline (speedup 1.0000x reference)
import jax
import jax.numpy as jnp
from jax.experimental import pallas as pl


def kernel(predictions, gt_boxes, gt_labels, priors):
    raise NotImplementedError("write your pallas kernel here")


# speedup vs baseline: 1.9338x; 1.9338x over previous
import functools

import jax
import jax.numpy as jnp
from jax import lax
from jax.experimental import pallas as pl
from jax.experimental.pallas import tpu as pltpu
from jax.experimental.pallas import tpu_sc as plsc

N = 10647
NOUT = 85
G = 32
NW = 16
CH = 672
NPAD = NW * CH
PV = CH // 16
POS_IOU = 0.7
NEG_IOU = 0.3
N_SAMPLE = 256.0
N_POS_CAP = 128.0


def _sc_match(pri, gtb, gtg):
    mesh = plsc.VectorSubcoreMesh(core_axis_name="c", subcore_axis_name="s",
                                  num_cores=1, num_subcores=NW)

    @functools.partial(
        pl.kernel,
        out_type=jax.ShapeDtypeStruct((NPAD * 8,), jnp.float32),
        mesh=mesh,
        compiler_params=pltpu.CompilerParams(needs_layout_passes=False),
        scratch_types=dict(
            priv=pltpu.VMEM((4, CH), jnp.float32),
            gtbv=pltpu.VMEM((5, G, 16), jnp.float32),
            gtgv=pltpu.VMEM((8, G), jnp.float32),
            pmaxv=pltpu.VMEM((CH,), jnp.float32),
            pidxv=pltpu.VMEM((CH,), jnp.int32),
            labv=pltpu.VMEM((CH,), jnp.float32),
            gmaxv=pltpu.VMEM((G, 16), jnp.float32),
            gidxv=pltpu.VMEM((G, 16), jnp.int32),
            locv=pltpu.VMEM((80,), jnp.float32),
            allv=pltpu.VMEM((NW, 80), jnp.float32),
            loc2=pltpu.VMEM((16,), jnp.float32),
            all2=pltpu.VMEM((NW, 16), jnp.float32),
            outv=pltpu.VMEM((CH * 8,), jnp.float32),
            sh1=pltpu.VMEM_SHARED((NW, 80), jnp.float32),
            sh2=pltpu.VMEM_SHARED((NW, 16), jnp.float32),
        ),
    )
    def k(pri_hbm, gtb_hbm, gtg_hbm, out_hbm, *, priv, gtbv, gtgv, pmaxv,
          pidxv, labv, gmaxv, gidxv, locv, allv, loc2, all2, outv, sh1, sh2):
        w = lax.axis_index("s")
        base = w * CH
        lane = lax.iota(jnp.int32, 16)
        ones = jnp.full((16,), 1.0, jnp.float32)

        pltpu.sync_copy(pri_hbm.at[w], priv)
        pltpu.sync_copy(gtb_hbm, gtbv)
        pltpu.sync_copy(gtg_hbm, gtgv)

        def init_g(j, _):
            gmaxv[j] = jnp.full((16,), -1.0, jnp.float32)
            gidxv[j] = jnp.zeros((16,), jnp.int32)
            return 0
        lax.fori_loop(0, G, init_g, 0)

        def p1(i, _):
            cx = priv[0, pl.ds(i * 16, 16)]
            cy = priv[1, pl.ds(i * 16, 16)]
            ww = priv[2, pl.ds(i * 16, 16)]
            hh = priv[3, pl.ds(i * 16, 16)]
            ax1 = cx - ww / 2.0
            ay1 = cy - hh / 2.0
            ax2 = cx + ww / 2.0
            ay2 = cy + hh / 2.0
            area_a = (ax2 - ax1) * (ay2 - ay1)
            gl = base + i * 16 + lane
            valid = gl < N

            def jbody(j, c):
                pmax, pidx = c
                bx1 = gtbv[0, j]
                by1 = gtbv[1, j]
                bx2 = gtbv[2, j]
                by2 = gtbv[3, j]
                ab = gtbv[4, j]
                iw = jnp.maximum(jnp.minimum(ax2, bx2) - jnp.maximum(ax1, bx1), 0.0)
                ih = jnp.maximum(jnp.minimum(ay2, by2) - jnp.maximum(ay1, by1), 0.0)
                inter = iw * ih
                iou = inter / (area_a + ab - inter)
                upd = iou > pmax
                pmax = jnp.where(upd, iou, pmax)
                pidx = jnp.where(upd, j, pidx)
                iv = jnp.where(valid, iou, -1.0)
                gm = gmaxv[j]
                gu = iv > gm
                gmaxv[j] = jnp.where(gu, iv, gm)
                gidxv[j] = jnp.where(gu, gl, gidxv[j])
                return pmax, pidx

            pmax, pidx = lax.fori_loop(
                0, G, jbody,
                (jnp.full((16,), -1.0, jnp.float32), jnp.zeros((16,), jnp.int32)))
            lab = jnp.where(pmax < NEG_IOU, 0.0,
                            jnp.where(pmax >= POS_IOU, 1.0, -1.0))
            lab = jnp.where(valid, lab, -1.0)
            labv[pl.ds(i * 16, 16)] = lab
            pmaxv[pl.ds(i * 16, 16)] = pmax
            pidxv[pl.ds(i * 16, 16)] = pidx
            return 0
        lax.fori_loop(0, PV, p1, 0)

        def p2a(j, _):
            gm = gmaxv[j]
            gi = gidxv[j]
            m = jnp.max(gm)
            mi = jnp.min(jnp.where(gm == m, gi, jnp.int32(2 ** 30)))
            plsc.store_scatter(locv, [jnp.full((16,), j, jnp.int32)],
                               jnp.full((16,), m, jnp.float32),
                               mask=lane == 0)
            plsc.store_scatter(locv, [jnp.full((16,), G + j, jnp.int32)],
                               jnp.full((16,), mi.astype(jnp.float32)),
                               mask=lane == 0)
            return 0
        lax.fori_loop(0, G, p2a, 0)

        pltpu.sync_copy(locv, sh1.at[w])
        plsc.subcore_barrier()
        pltpu.sync_copy(sh1, allv)

        def p2b(wp, c):
            va, vb, ia, ib = c
            v1 = allv[wp, pl.ds(0, 16)]
            v2 = allv[wp, pl.ds(16, 16)]
            i1 = allv[wp, pl.ds(32, 16)]
            i2 = allv[wp, pl.ds(48, 16)]
            u1 = v1 > va
            u2 = v2 > vb
            return (jnp.where(u1, v1, va), jnp.where(u2, v2, vb),
                    jnp.where(u1, i1, ia), jnp.where(u2, i2, ib))
        neg2 = jnp.full((16,), -2.0, jnp.float32)
        zf = jnp.zeros((16,), jnp.float32)
        _, _, ia, ib = lax.fori_loop(0, NW, p2b, (neg2, neg2, zf, zf))

        for bi in (ia, ib):
            gidx = bi.astype(jnp.int32) - base
            inr = (gidx >= 0) & (gidx < CH)
            gidx_c = jnp.where(inr, gidx, 0)
            pmv = plsc.load_gather(pmaxv, [gidx_c], mask=inr)
            cond = inr & (pmv >= NEG_IOU)
            plsc.store_scatter(labv, [gidx_c], ones, mask=cond)

        def cnt(i, c):
            pc, nc = c
            l = labv[pl.ds(i * 16, 16)]
            pc = pc + jnp.where(l == 1.0, 1.0, 0.0)
            nc = nc + jnp.where(l == 0.0, 1.0, 0.0)
            return pc, nc
        pc, nc = lax.fori_loop(0, PV, cnt, (zf, zf))
        pcs = jnp.sum(pc)
        ncs = jnp.sum(nc)
        plsc.store_scatter(loc2, [jnp.zeros((16,), jnp.int32)],
                           jnp.full((16,), pcs), mask=lane == 0)
        plsc.store_scatter(loc2, [jnp.full((16,), 1, jnp.int32)],
                           jnp.full((16,), ncs), mask=lane == 0)
        pltpu.sync_copy(loc2, sh2.at[w])
        plsc.subcore_barrier()
        pltpu.sync_copy(sh2, all2)

        def p2c(wp, c):
            tot, pre = c
            v = all2[wp]
            f = jnp.where(wp < w, 1.0, 0.0)
            return tot + v, pre + v * f
        tot, pre = lax.fori_loop(0, NW, p2c, (zf, zf))
        lane0 = lane == 0
        lane1 = lane == 1
        p_tot = jnp.sum(jnp.where(lane0, tot, 0.0))
        n_tot = jnp.sum(jnp.where(lane1, tot, 0.0))
        p_pre = jnp.sum(jnp.where(lane0, pre, 0.0))
        n_pre = jnp.sum(jnp.where(lane1, pre, 0.0))
        excess_pos = p_tot - N_POS_CAP
        n_pos_final = p_tot - jnp.maximum(0.0, excess_pos)
        excess_neg = n_tot - (N_SAMPLE - n_pos_final)

        def p3(i, c):
            cpos, cneg = c
            l = labv[pl.ds(i * 16, 16)]
            posf = jnp.where(l == 1.0, 1.0, 0.0)
            negf = jnp.where(l == 0.0, 1.0, 0.0)
            epp = plsc.cumsum(posf) - posf + cpos
            epn = plsc.cumsum(negf) - negf + cneg
            objf = jnp.where((posf > 0.0) & (epp >= excess_pos), 1.0, 0.0)
            nonf = jnp.where((negf > 0.0) & (epn >= excess_neg), 1.0, 0.0)
            pidx = pidxv[pl.ds(i * 16, 16)]
            t0 = plsc.load_gather(gtgv, [jnp.zeros((16,), jnp.int32), pidx])
            t1 = plsc.load_gather(gtgv, [jnp.full((16,), 1, jnp.int32), pidx])
            t2 = plsc.load_gather(gtgv, [jnp.full((16,), 2, jnp.int32), pidx])
            t3 = plsc.load_gather(gtgv, [jnp.full((16,), 3, jnp.int32), pidx])
            cf = plsc.load_gather(gtgv, [jnp.full((16,), 4, jnp.int32), pidx])
            li8 = (i * 16 + lane) * 8
            plsc.store_scatter(outv, [li8], objf)
            plsc.store_scatter(outv, [li8 + 1], nonf)
            plsc.store_scatter(outv, [li8 + 2], t0)
            plsc.store_scatter(outv, [li8 + 3], t1)
            plsc.store_scatter(outv, [li8 + 4], t2)
            plsc.store_scatter(outv, [li8 + 5], t3)
            plsc.store_scatter(outv, [li8 + 6], cf)
            plsc.store_scatter(outv, [li8 + 7], zf)
            return cpos + jnp.sum(posf), cneg + jnp.sum(negf)
        lax.fori_loop(0, PV, p3, (p_pre, n_pre))

        pltpu.sync_copy(outv, out_hbm.at[pl.ds(base * 8, CH * 8)])

    return k(pri, gtb, gtg)


def _tc_body(pred_ref, aux_ref, out_ref, acc):
    step = pl.program_id(0)

    @pl.when(step == 0)
    def _():
        for t in range(6):
            acc[t] = 0.0

    rblk = pred_ref.shape[0]
    rowid = step * rblk + lax.broadcasted_iota(jnp.int32, (rblk, 1), 0)
    valid = rowid < N
    p = jnp.where(valid, pred_ref[...], 0.0)
    obj = aux_ref[:, 0:1]
    nob = aux_ref[:, 1:2]
    tb = aux_ref[:, 2:6]
    col = aux_ref[:, 6:7].astype(jnp.int32)

    d = p[:, 0:4] - tb
    a_blk = jnp.sum(obj * (d * d))
    l4 = p[:, 4:5]
    sp4 = jnp.maximum(l4, 0.0) + jnp.log1p(jnp.exp(-jnp.abs(l4)))
    b_blk = jnp.sum(obj * (sp4 - l4))
    c_blk = jnp.sum(nob * sp4)
    cls = p[:, 5:NOUT]
    spc = jnp.maximum(cls, 0.0) + jnp.log1p(jnp.exp(-jnp.abs(cls)))
    s_blk = jnp.sum(obj * spc)
    lanes = lax.broadcasted_iota(jnp.int32, (rblk, NOUT), 1)
    pcol = jnp.sum(obj * jnp.where(lanes == col, p, 0.0))
    d_blk = s_blk - pcol

    acc[0] += a_blk
    acc[1] += b_blk
    acc[2] += c_blk
    acc[3] += d_blk
    acc[4] += jnp.sum(obj)
    acc[5] += jnp.sum(nob)

    @pl.when(step == pl.num_programs(0) - 1)
    def _():
        n_obj = acc[4]
        n_non = acc[5]
        total = ((acc[0] + acc[1]) / n_obj + acc[2] / n_non
                 + acc[3] / (n_obj * 80.0))
        out_ref[...] = jnp.full((1, 1), total, jnp.float32)


def _tc_loss(pred, aux):
    rblk = 1344
    grid = NPAD // rblk
    return pl.pallas_call(
        _tc_body,
        grid=(grid,),
        in_specs=[
            pl.BlockSpec((rblk, NOUT), lambda i: (i, 0)),
            pl.BlockSpec((rblk, 8), lambda i: (i, 0)),
        ],
        out_specs=pl.BlockSpec((1, 1), lambda i: (0, 0)),
        out_shape=jax.ShapeDtypeStruct((1, 1), jnp.float32),
        scratch_shapes=[pltpu.SMEM((8,), jnp.float32)],
    )(pred, aux)


def kernel(predictions, gt_boxes, gt_labels, priors):
    pred = predictions[0]

    priT = priors[:, :4].T
    priT = jnp.pad(priT, ((0, 0), (0, NPAD - N)))
    pri = priT.reshape(4, NW, CH).transpose(1, 0, 2)

    gx1 = gt_boxes[:, 0] - gt_boxes[:, 2] / 2.0
    gy1 = gt_boxes[:, 1] - gt_boxes[:, 3] / 2.0
    gx2 = gt_boxes[:, 0] + gt_boxes[:, 2] / 2.0
    gy2 = gt_boxes[:, 1] + gt_boxes[:, 3] / 2.0
    area_b = (gx2 - gx1) * (gy2 - gy1)
    gtb = jnp.broadcast_to(
        jnp.stack([gx1, gy1, gx2, gy2, area_b])[:, :, None], (5, G, 16)) + 0.0

    colf = (gt_labels + 5).astype(jnp.float32)
    gtg = jnp.concatenate(
        [gt_boxes.T, colf[None, :], jnp.zeros((3, G), jnp.float32)], axis=0)

    aux = _sc_match(pri, gtb, gtg).reshape(NPAD, 8)
    total = _tc_loss(pred, aux)[0, 0]
    return pred, total


# speedup vs baseline: 2.3954x; 1.2387x over previous
import functools

import jax
import jax.numpy as jnp
from jax import lax
from jax.experimental import pallas as pl
from jax.experimental.pallas import tpu as pltpu
from jax.experimental.pallas import tpu_sc as plsc

N = 10647
NOUT = 85
G = 32
NW = 16
CH = 672
NPAD = NW * CH
PV = CH // 16
POS_IOU = 0.7
NEG_IOU = 0.3
N_SAMPLE = 256.0
N_POS_CAP = 128.0


def _sc_match(pri, gtb, gtg):
    mesh = plsc.VectorSubcoreMesh(core_axis_name="c", subcore_axis_name="s",
                                  num_cores=1, num_subcores=NW)

    @functools.partial(
        pl.kernel,
        out_type=jax.ShapeDtypeStruct((NPAD * 8,), jnp.float32),
        mesh=mesh,
        compiler_params=pltpu.CompilerParams(needs_layout_passes=False),
        scratch_types=dict(
            priv=pltpu.VMEM((4, CH), jnp.float32),
            crn=pltpu.VMEM((5, CH), jnp.float32),
            gtbv=pltpu.VMEM((5, G, 16), jnp.float32),
            gtgv=pltpu.VMEM((8, G), jnp.float32),
            pmaxv=pltpu.VMEM((CH,), jnp.float32),
            pidxv=pltpu.VMEM((CH,), jnp.int32),
            labv=pltpu.VMEM((CH,), jnp.float32),
            gmaxv=pltpu.VMEM((G, 16), jnp.float32),
            gidxv=pltpu.VMEM((G, 16), jnp.int32),
            locv=pltpu.VMEM((80,), jnp.float32),
            allv=pltpu.VMEM((NW, 80), jnp.float32),
            loc2=pltpu.VMEM((16,), jnp.int32),
            all2=pltpu.VMEM((NW, 16), jnp.int32),
            outv=pltpu.VMEM((CH * 8,), jnp.float32),
            sh1=pltpu.VMEM_SHARED((NW, 80), jnp.float32),
            sh2=pltpu.VMEM_SHARED((NW, 16), jnp.int32),
        ),
    )
    def k(pri_hbm, gtb_hbm, gtg_hbm, out_hbm, *, priv, crn, gtbv, gtgv, pmaxv,
          pidxv, labv, gmaxv, gidxv, locv, allv, loc2, all2, outv, sh1, sh2):
        w = lax.axis_index("s")
        base = w * CH
        lane = lax.iota(jnp.int32, 16)
        ones = jnp.full((16,), 1.0, jnp.float32)

        pltpu.sync_copy(pri_hbm.at[w], priv)
        pltpu.sync_copy(gtb_hbm, gtbv)
        pltpu.sync_copy(gtg_hbm, gtgv)

        def p1a(i, _):
            sl = pl.ds(i * 16, 16)
            cx = priv[0, sl]
            cy = priv[1, sl]
            ww = priv[2, sl]
            hh = priv[3, sl]
            ax1 = cx - ww / 2.0
            ay1 = cy - hh / 2.0
            ax2 = cx + ww / 2.0
            ay2 = cy + hh / 2.0
            crn[0, sl] = ax1
            crn[1, sl] = ay1
            crn[2, sl] = ax2
            crn[3, sl] = ay2
            crn[4, sl] = (ax2 - ax1) * (ay2 - ay1)
            pmaxv[sl] = jnp.full((16,), -1.0, jnp.float32)
            pidxv[sl] = jnp.zeros((16,), jnp.int32)
            return 0
        lax.fori_loop(0, PV, p1a, 0)

        JB = 4
        def p1b(jb, _):
            j0 = jb * JB
            gt = [[gtbv[kf, j0 + u] for kf in range(5)] for u in range(JB)]

            def strip(i, c):
                sl = pl.ds(i * 16, 16)
                ax1 = crn[0, sl]
                ay1 = crn[1, sl]
                ax2 = crn[2, sl]
                ay2 = crn[3, sl]
                area_a = crn[4, sl]
                pmax = pmaxv[sl]
                pidx = pidxv[sl]
                gl = base + i * 16 + lane
                gms, gis = list(c[0]), list(c[1])
                for u in range(JB):
                    bx1, by1, bx2, by2, ab = gt[u]
                    iw = jnp.maximum(
                        jnp.minimum(ax2, bx2) - jnp.maximum(ax1, bx1), 0.0)
                    ih = jnp.maximum(
                        jnp.minimum(ay2, by2) - jnp.maximum(ay1, by1), 0.0)
                    inter = iw * ih
                    iou = inter / (area_a + ab - inter)
                    upd = iou > pmax
                    pmax = jnp.where(upd, iou, pmax)
                    pidx = jnp.where(upd, j0 + u, pidx)
                    gu = iou > gms[u]
                    gms[u] = jnp.where(gu, iou, gms[u])
                    gis[u] = jnp.where(gu, gl, gis[u])
                pmaxv[sl] = pmax
                pidxv[sl] = pidx
                return tuple(gms), tuple(gis)

            gm0 = tuple(jnp.full((16,), -1.0, jnp.float32) for _ in range(JB))
            gi0 = tuple(jnp.zeros((16,), jnp.int32) for _ in range(JB))
            gms, gis = lax.fori_loop(0, PV, strip, (gm0, gi0))
            for u in range(JB):
                gmaxv[j0 + u] = gms[u]
                gidxv[j0 + u] = gis[u]
            return 0
        lax.fori_loop(0, G // JB, p1b, 0)

        def p1c(i, _):
            sl = pl.ds(i * 16, 16)
            pmax = pmaxv[sl]
            valid = (base + i * 16 + lane) < N
            lab = jnp.where(pmax < NEG_IOU, 0.0,
                            jnp.where(pmax >= POS_IOU, 1.0, -1.0))
            labv[sl] = jnp.where(valid, lab, -1.0)
            return 0
        lax.fori_loop(0, PV, p1c, 0)

        def p2a(j, c):
            va, vb, ja, jb_ = c
            gm = gmaxv[j]
            gi = gidxv[j]
            m = jnp.max(gm)
            mi = jnp.min(jnp.where(gm == m, gi, jnp.int32(2 ** 30)))
            mf = mi.astype(jnp.float32)
            va = va + jnp.where(lane == j, m, 0.0)
            vb = vb + jnp.where(lane == j - 16, m, 0.0)
            ja = ja + jnp.where(lane == j, mf, 0.0)
            jb_ = jb_ + jnp.where(lane == j - 16, mf, 0.0)
            return va, vb, ja, jb_
        zf16 = jnp.zeros((16,), jnp.float32)
        va, vb, ja_, jb_ = lax.fori_loop(0, G, p2a, (zf16, zf16, zf16, zf16))
        locv[pl.ds(0, 16)] = va
        locv[pl.ds(16, 16)] = vb
        locv[pl.ds(32, 16)] = ja_
        locv[pl.ds(48, 16)] = jb_

        pltpu.sync_copy(locv, sh1.at[w])
        plsc.subcore_barrier()
        pltpu.sync_copy(sh1, allv)

        def p2b(wp, c):
            va, vb, ia, ib = c
            v1 = allv[wp, pl.ds(0, 16)]
            v2 = allv[wp, pl.ds(16, 16)]
            i1 = allv[wp, pl.ds(32, 16)]
            i2 = allv[wp, pl.ds(48, 16)]
            u1 = v1 > va
            u2 = v2 > vb
            return (jnp.where(u1, v1, va), jnp.where(u2, v2, vb),
                    jnp.where(u1, i1, ia), jnp.where(u2, i2, ib))
        neg2 = jnp.full((16,), -2.0, jnp.float32)
        zf = jnp.zeros((16,), jnp.float32)
        _, _, ia, ib = lax.fori_loop(0, NW, p2b, (neg2, neg2, zf, zf))

        for bi in (ia, ib):
            gidx = bi.astype(jnp.int32) - base
            inr = (gidx >= 0) & (gidx < CH)
            gidx_c = jnp.where(inr, gidx, 0)
            pmv = plsc.load_gather(pmaxv, [gidx_c], mask=inr)
            cond = inr & (pmv >= NEG_IOU)
            plsc.store_scatter(labv, [gidx_c], ones, mask=cond)

        zi = jnp.zeros((16,), jnp.int32)
        def cnt(i, c):
            pc, nc = c
            l = labv[pl.ds(i * 16, 16)]
            pc = pc + jnp.where(l == 1.0, 1, 0)
            nc = nc + jnp.where(l == 0.0, 1, 0)
            return pc, nc
        pc, nc = lax.fori_loop(0, PV, cnt, (zi, zi))
        combo = (jnp.sum(pc) << 16) | jnp.sum(nc)
        loc2[...] = jnp.where(lane == 0, combo, 0)
        pltpu.sync_copy(loc2, sh2.at[w])
        plsc.subcore_barrier()
        pltpu.sync_copy(sh2, all2)

        def p2c(wp, c):
            tot, pre = c
            v = all2[wp]
            return tot + v, pre + v * jnp.where(wp < w, 1, 0)
        tot, pre = lax.fori_loop(0, NW, p2c, (zi, zi))
        lane0 = lane == 0
        tot_c = jnp.sum(jnp.where(lane0, tot, 0))
        pre_c = jnp.sum(jnp.where(lane0, pre, 0))
        p_tot = (tot_c >> 16).astype(jnp.float32)
        n_tot = (tot_c & 0xFFFF).astype(jnp.float32)
        p_pre = (pre_c >> 16).astype(jnp.float32)
        n_pre = (pre_c & 0xFFFF).astype(jnp.float32)
        excess_pos = p_tot - N_POS_CAP
        n_pos_final = p_tot - jnp.maximum(0.0, excess_pos)
        excess_neg = n_tot - (N_SAMPLE - n_pos_final)

        def p3(i, c):
            cpos, cneg = c
            l = labv[pl.ds(i * 16, 16)]
            posf = jnp.where(l == 1.0, 1.0, 0.0)
            negf = jnp.where(l == 0.0, 1.0, 0.0)
            epp = plsc.cumsum(posf) - posf + cpos
            epn = plsc.cumsum(negf) - negf + cneg
            objf = jnp.where((posf > 0.0) & (epp >= excess_pos), 1.0, 0.0)
            nonf = jnp.where((negf > 0.0) & (epn >= excess_neg), 1.0, 0.0)
            pidx = pidxv[pl.ds(i * 16, 16)]
            t0 = plsc.load_gather(gtgv, [jnp.zeros((16,), jnp.int32), pidx])
            t1 = plsc.load_gather(gtgv, [jnp.full((16,), 1, jnp.int32), pidx])
            t2 = plsc.load_gather(gtgv, [jnp.full((16,), 2, jnp.int32), pidx])
            t3 = plsc.load_gather(gtgv, [jnp.full((16,), 3, jnp.int32), pidx])
            cf = plsc.load_gather(gtgv, [jnp.full((16,), 4, jnp.int32), pidx])
            li8 = (i * 16 + lane) * 8
            plsc.store_scatter(outv, [li8], objf)
            plsc.store_scatter(outv, [li8 + 1], nonf)
            plsc.store_scatter(outv, [li8 + 2], t0)
            plsc.store_scatter(outv, [li8 + 3], t1)
            plsc.store_scatter(outv, [li8 + 4], t2)
            plsc.store_scatter(outv, [li8 + 5], t3)
            plsc.store_scatter(outv, [li8 + 6], cf)
            plsc.store_scatter(outv, [li8 + 7], zf)
            return cpos + jnp.sum(posf), cneg + jnp.sum(negf)
        lax.fori_loop(0, PV, p3, (p_pre, n_pre))

        pltpu.sync_copy(outv, out_hbm.at[pl.ds(base * 8, CH * 8)])

    return k(pri, gtb, gtg)


def _tc_body(pred_ref, aux_ref, copy_ref, out_ref, acc):
    step = pl.program_id(0)

    @pl.when(step == 0)
    def _():
        for t in range(6):
            acc[t] = 0.0

    rblk = pred_ref.shape[0]
    rowid = step * rblk + lax.broadcasted_iota(jnp.int32, (rblk, 1), 0)
    valid = rowid < N
    raw = pred_ref[...]
    copy_ref[...] = raw
    p = jnp.where(valid, raw, 0.0)
    obj = aux_ref[:, 0:1]
    nob = aux_ref[:, 1:2]
    tb = aux_ref[:, 2:6]
    col = aux_ref[:, 6:7].astype(jnp.int32)

    d = p[:, 0:4] - tb
    a_blk = jnp.sum(obj * (d * d))
    l4 = p[:, 4:5]
    sp4 = jnp.maximum(l4, 0.0) + jnp.log1p(jnp.exp(-jnp.abs(l4)))
    b_blk = jnp.sum(obj * (sp4 - l4))
    c_blk = jnp.sum(nob * sp4)
    cls = p[:, 5:NOUT]
    spc = jnp.maximum(cls, 0.0) + jnp.log1p(jnp.exp(-jnp.abs(cls)))
    s_blk = jnp.sum(obj * spc)
    lanes = lax.broadcasted_iota(jnp.int32, (rblk, NOUT), 1)
    pcol = jnp.sum(obj * jnp.where(lanes == col, p, 0.0))
    d_blk = s_blk - pcol

    acc[0] += a_blk
    acc[1] += b_blk
    acc[2] += c_blk
    acc[3] += d_blk
    acc[4] += jnp.sum(obj)
    acc[5] += jnp.sum(nob)

    @pl.when(step == pl.num_programs(0) - 1)
    def _():
        n_obj = acc[4]
        n_non = acc[5]
        total = ((acc[0] + acc[1]) / n_obj + acc[2] / n_non
                 + acc[3] / (n_obj * 80.0))
        out_ref[...] = jnp.full((1, 1), total, jnp.float32)


def _tc_loss(pred, aux):
    rblk = 1344
    grid = NPAD // rblk
    return pl.pallas_call(
        _tc_body,
        grid=(grid,),
        in_specs=[
            pl.BlockSpec((rblk, NOUT), lambda i: (i, 0)),
            pl.BlockSpec((rblk, 8), lambda i: (i, 0)),
        ],
        out_specs=[
            pl.BlockSpec((rblk, NOUT), lambda i: (i, 0)),
            pl.BlockSpec((1, 1), lambda i: (0, 0)),
        ],
        out_shape=[
            jax.ShapeDtypeStruct((N, NOUT), jnp.float32),
            jax.ShapeDtypeStruct((1, 1), jnp.float32),
        ],
        scratch_shapes=[pltpu.SMEM((8,), jnp.float32)],
    )(pred, aux)


def kernel(predictions, gt_boxes, gt_labels, priors):
    pred = predictions[0]

    priT = priors[:, :4].T
    priT = jnp.pad(priT, ((0, 0), (0, NPAD - N)))
    pri = priT.reshape(4, NW, CH).transpose(1, 0, 2)

    gx1 = gt_boxes[:, 0] - gt_boxes[:, 2] / 2.0
    gy1 = gt_boxes[:, 1] - gt_boxes[:, 3] / 2.0
    gx2 = gt_boxes[:, 0] + gt_boxes[:, 2] / 2.0
    gy2 = gt_boxes[:, 1] + gt_boxes[:, 3] / 2.0
    area_b = (gx2 - gx1) * (gy2 - gy1)
    gtb = jnp.broadcast_to(
        jnp.stack([gx1, gy1, gx2, gy2, area_b])[:, :, None], (5, G, 16)) + 0.0

    colf = (gt_labels + 5).astype(jnp.float32)
    gtg = jnp.concatenate(
        [gt_boxes.T, colf[None, :], jnp.zeros((3, G), jnp.float32)], axis=0)

    aux = _sc_match(pri, gtb, gtg).reshape(NPAD, 8)
    pred_copy, tot = _tc_loss(pred, aux)
    return pred_copy, tot[0, 0]


# speedup vs baseline: 2.3968x; 1.0006x over previous
import functools

import jax
import jax.numpy as jnp
from jax import lax
from jax.experimental import pallas as pl
from jax.experimental.pallas import tpu as pltpu
from jax.experimental.pallas import tpu_sc as plsc

N = 10647
NOUT = 85
G = 32
NW = 16
CH = 672
NPAD = NW * CH
PV = CH // 16
POS_IOU = 0.7
NEG_IOU = 0.3
N_SAMPLE = 256.0
N_POS_CAP = 128.0


def _sc_match(pri, gtb, gtg):
    mesh = plsc.VectorSubcoreMesh(core_axis_name="c", subcore_axis_name="s",
                                  num_cores=1, num_subcores=NW)

    @functools.partial(
        pl.kernel,
        out_type=jax.ShapeDtypeStruct((NPAD * 8,), jnp.float32),
        mesh=mesh,
        compiler_params=pltpu.CompilerParams(needs_layout_passes=False),
        scratch_types=dict(
            priv=pltpu.VMEM((4, CH), jnp.float32),
            crn=pltpu.VMEM((5, CH), jnp.float32),
            gtbv=pltpu.VMEM((5, G, 16), jnp.float32),
            gtgv=pltpu.VMEM((8, G), jnp.float32),
            pmaxv=pltpu.VMEM((CH,), jnp.float32),
            pidxv=pltpu.VMEM((CH,), jnp.int32),
            labv=pltpu.VMEM((CH,), jnp.float32),
            gmaxv=pltpu.VMEM((G, 16), jnp.float32),
            gidxv=pltpu.VMEM((G, 16), jnp.int32),
            locv=pltpu.VMEM((80,), jnp.float32),
            allv=pltpu.VMEM((NW, 80), jnp.float32),
            loc2=pltpu.VMEM((16,), jnp.int32),
            all2=pltpu.VMEM((NW, 16), jnp.int32),
            outv=pltpu.VMEM((CH * 8,), jnp.float32),
            sh1=pltpu.VMEM_SHARED((NW, 80), jnp.float32),
            sh2=pltpu.VMEM_SHARED((NW, 16), jnp.int32),
        ),
    )
    def k(pri_hbm, gtb_hbm, gtg_hbm, out_hbm, *, priv, crn, gtbv, gtgv, pmaxv,
          pidxv, labv, gmaxv, gidxv, locv, allv, loc2, all2, outv, sh1, sh2):
        w = lax.axis_index("s")
        base = w * CH
        lane = lax.iota(jnp.int32, 16)
        ones = jnp.full((16,), 1.0, jnp.float32)

        pltpu.sync_copy(pri_hbm.at[w], priv)
        pltpu.sync_copy(gtb_hbm, gtbv)
        pltpu.sync_copy(gtg_hbm, gtgv)

        def p1a(i, _):
            sl = pl.ds(i * 16, 16)
            cx = priv[0, sl]
            cy = priv[1, sl]
            ww = priv[2, sl]
            hh = priv[3, sl]
            ax1 = cx - ww / 2.0
            ay1 = cy - hh / 2.0
            ax2 = cx + ww / 2.0
            ay2 = cy + hh / 2.0
            crn[0, sl] = ax1
            crn[1, sl] = ay1
            crn[2, sl] = ax2
            crn[3, sl] = ay2
            crn[4, sl] = (ax2 - ax1) * (ay2 - ay1)
            pmaxv[sl] = jnp.full((16,), -1.0, jnp.float32)
            pidxv[sl] = jnp.zeros((16,), jnp.int32)
            return 0
        lax.fori_loop(0, PV, p1a, 0)

        JB = 4
        def p1b(jb, _):
            j0 = jb * JB
            gt = [[gtbv[kf, j0 + u] for kf in range(5)] for u in range(JB)]

            def strip(i, c):
                sl = pl.ds(i * 16, 16)
                ax1 = crn[0, sl]
                ay1 = crn[1, sl]
                ax2 = crn[2, sl]
                ay2 = crn[3, sl]
                area_a = crn[4, sl]
                pmax = pmaxv[sl]
                pidx = pidxv[sl]
                gl = base + i * 16 + lane
                gms, gis = list(c[0]), list(c[1])
                for u in range(JB):
                    bx1, by1, bx2, by2, ab = gt[u]
                    iw = jnp.maximum(
                        jnp.minimum(ax2, bx2) - jnp.maximum(ax1, bx1), 0.0)
                    ih = jnp.maximum(
                        jnp.minimum(ay2, by2) - jnp.maximum(ay1, by1), 0.0)
                    inter = iw * ih
                    iou = inter / (area_a + ab - inter)
                    upd = iou > pmax
                    pmax = jnp.where(upd, iou, pmax)
                    pidx = jnp.where(upd, j0 + u, pidx)
                    gu = iou > gms[u]
                    gms[u] = jnp.where(gu, iou, gms[u])
                    gis[u] = jnp.where(gu, gl, gis[u])
                pmaxv[sl] = pmax
                pidxv[sl] = pidx
                return tuple(gms), tuple(gis)

            gm0 = tuple(jnp.full((16,), -1.0, jnp.float32) for _ in range(JB))
            gi0 = tuple(jnp.zeros((16,), jnp.int32) for _ in range(JB))
            gms, gis = lax.fori_loop(0, PV, strip, (gm0, gi0))
            for u in range(JB):
                gmaxv[j0 + u] = gms[u]
                gidxv[j0 + u] = gis[u]
            return 0
        lax.fori_loop(0, G // JB, p1b, 0)

        def p1c(i, _):
            sl = pl.ds(i * 16, 16)
            pmax = pmaxv[sl]
            valid = (base + i * 16 + lane) < N
            lab = jnp.where(pmax < NEG_IOU, 0.0,
                            jnp.where(pmax >= POS_IOU, 1.0, -1.0))
            labv[sl] = jnp.where(valid, lab, -1.0)
            return 0
        lax.fori_loop(0, PV, p1c, 0)

        def p2a(j, c):
            va, vb, ja, jb_ = c
            gm = gmaxv[j]
            gi = gidxv[j]
            m = jnp.max(gm)
            mi = jnp.min(jnp.where(gm == m, gi, jnp.int32(2 ** 30)))
            mf = mi.astype(jnp.float32)
            va = va + jnp.where(lane == j, m, 0.0)
            vb = vb + jnp.where(lane == j - 16, m, 0.0)
            ja = ja + jnp.where(lane == j, mf, 0.0)
            jb_ = jb_ + jnp.where(lane == j - 16, mf, 0.0)
            return va, vb, ja, jb_
        zf = jnp.zeros((16,), jnp.float32)
        va, vb, ja_, jb_ = lax.fori_loop(0, G, p2a, (zf, zf, zf, zf))
        locv[pl.ds(0, 16)] = va
        locv[pl.ds(16, 16)] = vb
        locv[pl.ds(32, 16)] = ja_
        locv[pl.ds(48, 16)] = jb_

        pltpu.sync_copy(locv, sh1.at[w])
        plsc.subcore_barrier()
        pltpu.sync_copy(sh1, allv)

        def p2b(wp, c):
            va, vb, ia, ib = c
            v1 = allv[wp, pl.ds(0, 16)]
            v2 = allv[wp, pl.ds(16, 16)]
            i1 = allv[wp, pl.ds(32, 16)]
            i2 = allv[wp, pl.ds(48, 16)]
            u1 = v1 > va
            u2 = v2 > vb
            return (jnp.where(u1, v1, va), jnp.where(u2, v2, vb),
                    jnp.where(u1, i1, ia), jnp.where(u2, i2, ib))
        neg2 = jnp.full((16,), -2.0, jnp.float32)
        _, _, ia, ib = lax.fori_loop(0, NW, p2b, (neg2, neg2, zf, zf))

        for bi in (ia, ib):
            gidx = bi.astype(jnp.int32) - base
            inr = (gidx >= 0) & (gidx < CH)
            gidx_c = jnp.where(inr, gidx, 0)
            pmv = plsc.load_gather(pmaxv, [gidx_c], mask=inr)
            cond = inr & (pmv >= NEG_IOU)
            plsc.store_scatter(labv, [gidx_c], ones, mask=cond)

        zi = jnp.zeros((16,), jnp.int32)
        def cnt(i, c):
            pc, nc = c
            l = labv[pl.ds(i * 16, 16)]
            pc = pc + jnp.where(l == 1.0, 1, 0)
            nc = nc + jnp.where(l == 0.0, 1, 0)
            return pc, nc
        pc, nc = lax.fori_loop(0, PV, cnt, (zi, zi))
        combo = (jnp.sum(pc) << 16) | jnp.sum(nc)
        loc2[...] = jnp.where(lane == 0, combo, 0)
        pltpu.sync_copy(loc2, sh2.at[w])
        plsc.subcore_barrier()
        pltpu.sync_copy(sh2, all2)

        def p2c(wp, c):
            tot, pre = c
            v = all2[wp]
            return tot + v, pre + v * jnp.where(wp < w, 1, 0)
        tot, pre = lax.fori_loop(0, NW, p2c, (zi, zi))
        lane0 = lane == 0
        tot_c = jnp.sum(jnp.where(lane0, tot, 0))
        pre_c = jnp.sum(jnp.where(lane0, pre, 0))
        p_tot = (tot_c >> 16).astype(jnp.float32)
        n_tot = (tot_c & 0xFFFF).astype(jnp.float32)
        p_pre = (pre_c >> 16).astype(jnp.float32)
        n_pre = (pre_c & 0xFFFF).astype(jnp.float32)
        excess_pos = p_tot - N_POS_CAP
        n_pos_final = p_tot - jnp.maximum(0.0, excess_pos)
        excess_neg = n_tot - (N_SAMPLE - n_pos_final)

        def p3(i, c):
            cpos, cneg = c
            sl = pl.ds(i * 16, 16)
            l = labv[sl]
            posf = jnp.where(l == 1.0, 1.0, 0.0)
            negf = jnp.where(l == 0.0, 1.0, 0.0)
            epp = plsc.cumsum(posf) - posf + cpos
            epn = plsc.cumsum(negf) - negf + cneg
            objf = jnp.where((posf > 0.0) & (epp >= excess_pos), 1.0, 0.0)
            nonf = jnp.where((negf > 0.0) & (epn >= excess_neg), 1.0, 0.0)
            pidx = pidxv[sl]
            t0 = plsc.load_gather(gtgv, [jnp.zeros((16,), jnp.int32), pidx])
            t1 = plsc.load_gather(gtgv, [jnp.full((16,), 1, jnp.int32), pidx])
            t2 = plsc.load_gather(gtgv, [jnp.full((16,), 2, jnp.int32), pidx])
            t3 = plsc.load_gather(gtgv, [jnp.full((16,), 3, jnp.int32), pidx])
            cf = plsc.load_gather(gtgv, [jnp.full((16,), 4, jnp.int32), pidx])
            li8 = (i * 16 + lane) * 8
            plsc.store_scatter(outv, [li8], objf)
            plsc.store_scatter(outv, [li8 + 1], nonf)
            plsc.store_scatter(outv, [li8 + 2], t0)
            plsc.store_scatter(outv, [li8 + 3], t1)
            plsc.store_scatter(outv, [li8 + 4], t2)
            plsc.store_scatter(outv, [li8 + 5], t3)
            plsc.store_scatter(outv, [li8 + 6], cf)
            plsc.store_scatter(outv, [li8 + 7], zf)
            return cpos + jnp.sum(posf), cneg + jnp.sum(negf)
        lax.fori_loop(0, PV, p3, (p_pre, n_pre))

        pltpu.sync_copy(outv, out_hbm.at[pl.ds(base * 8, CH * 8)])

    return k(pri, gtb, gtg)


def _tc_body(pred_ref, aux_ref, copy_ref, out_ref, acc):
    step = pl.program_id(0)

    @pl.when(step == 0)
    def _():
        for t in range(6):
            acc[t] = 0.0

    rblk = pred_ref.shape[0]
    rowid = step * rblk + lax.broadcasted_iota(jnp.int32, (rblk, 1), 0)
    valid = rowid < N
    raw = pred_ref[...]
    copy_ref[...] = raw
    p = jnp.where(valid, raw, 0.0)
    obj = aux_ref[:, 0:1]
    nob = aux_ref[:, 1:2]
    tb = aux_ref[:, 2:6]
    col = aux_ref[:, 6:7].astype(jnp.int32)

    d = p[:, 0:4] - tb
    a_blk = jnp.sum(obj * (d * d))
    l4 = p[:, 4:5]
    sp4 = jnp.maximum(l4, 0.0) + jnp.log1p(jnp.exp(-jnp.abs(l4)))
    b_blk = jnp.sum(obj * (sp4 - l4))
    c_blk = jnp.sum(nob * sp4)
    cls = p[:, 5:NOUT]
    spc = jnp.maximum(cls, 0.0) + jnp.log1p(jnp.exp(-jnp.abs(cls)))
    s_blk = jnp.sum(obj * spc)
    lanes = lax.broadcasted_iota(jnp.int32, (rblk, NOUT), 1)
    pcol = jnp.sum(obj * jnp.where(lanes == col, p, 0.0))
    d_blk = s_blk - pcol

    acc[0] += a_blk
    acc[1] += b_blk
    acc[2] += c_blk
    acc[3] += d_blk
    acc[4] += jnp.sum(obj)
    acc[5] += jnp.sum(nob)

    @pl.when(step == pl.num_programs(0) - 1)
    def _():
        n_obj = acc[4]
        n_non = acc[5]
        total = ((acc[0] + acc[1]) / n_obj + acc[2] / n_non
                 + acc[3] / (n_obj * 80.0))
        out_ref[...] = jnp.full((1, 1), total, jnp.float32)


def _tc_loss(pred, aux):
    rblk = 1344
    grid = NPAD // rblk
    return pl.pallas_call(
        _tc_body,
        grid=(grid,),
        in_specs=[
            pl.BlockSpec((rblk, NOUT), lambda i: (i, 0)),
            pl.BlockSpec((rblk, 8), lambda i: (i, 0)),
        ],
        out_specs=[
            pl.BlockSpec((rblk, NOUT), lambda i: (i, 0)),
            pl.BlockSpec((1, 1), lambda i: (0, 0)),
        ],
        out_shape=[
            jax.ShapeDtypeStruct((N, NOUT), jnp.float32),
            jax.ShapeDtypeStruct((1, 1), jnp.float32),
        ],
        scratch_shapes=[pltpu.SMEM((8,), jnp.float32)],
    )(pred, aux)


def kernel(predictions, gt_boxes, gt_labels, priors):
    pred = predictions[0]

    priT = priors[:, :4].T
    priT = jnp.pad(priT, ((0, 0), (0, NPAD - N)))
    pri = priT.reshape(4, NW, CH).transpose(1, 0, 2)

    gx1 = gt_boxes[:, 0] - gt_boxes[:, 2] / 2.0
    gy1 = gt_boxes[:, 1] - gt_boxes[:, 3] / 2.0
    gx2 = gt_boxes[:, 0] + gt_boxes[:, 2] / 2.0
    gy2 = gt_boxes[:, 1] + gt_boxes[:, 3] / 2.0
    area_b = (gx2 - gx1) * (gy2 - gy1)
    gtb = jnp.broadcast_to(
        jnp.stack([gx1, gy1, gx2, gy2, area_b])[:, :, None], (5, G, 16)) + 0.0

    colf = (gt_labels + 5).astype(jnp.float32)
    gtg = jnp.concatenate(
        [gt_boxes.T, colf[None, :], jnp.zeros((3, G), jnp.float32)], axis=0)

    aux = _sc_match(pri, gtb, gtg).reshape(NPAD, 8)
    pred_copy, tot = _tc_loss(pred, aux)
    return pred_copy, tot[0, 0]
